# baseline (speedup 1.0000x reference)
import functools

import jax
import jax.numpy as jnp
from jax import lax
from jax.experimental import pallas as pl
from jax.experimental.pallas import tpu as pltpu
from jax.experimental.pallas import tpu_sc as plsc

_NC = 2
_NS = 16
_NW = _NC * _NS
_CHUNK = 128
_D = 16


def _sc_gather(table, idx3):
    NW, K, C = idx3.shape
    rows_per_w = K * C
    B_total = NW * rows_per_w
    mesh = plsc.VectorSubcoreMesh(core_axis_name="c", subcore_axis_name="s")

    @functools.partial(
        pl.kernel,
        mesh=mesh,
        compiler_params=pltpu.CompilerParams(use_tc_tiling_on_sc=False),
        out_type=jax.ShapeDtypeStruct((B_total, _D), jnp.float32),
        scratch_types=[
            pltpu.VMEM((K, C), jnp.int32),
            pltpu.VMEM((rows_per_w, _D), jnp.float32),
            pltpu.SemaphoreType.DMA,
        ],
    )
    def gather_k(table_hbm, idx_hbm, out_hbm, idx_v, rows_v, sem):
        wid = lax.axis_index("s") * _NC + lax.axis_index("c")
        pltpu.sync_copy(idx_hbm.at[wid], idx_v)
        cps = [
            pltpu.async_copy(
                table_hbm.at[idx_v.at[j]], rows_v.at[pl.ds(j * C, C)], sem
            )
            for j in range(K)
        ]
        for cp in cps:
            cp.wait()
        pltpu.sync_copy(rows_v, out_hbm.at[pl.ds(wid * rows_per_w, rows_per_w)])

    return gather_k(table, idx3)


def _mlp_body(h_ref, W1_ref, b1_ref, g1_ref, be1_ref, W2_ref, b2_ref,
              g2_ref, be2_ref, Wfc_ref, bfc_ref, out_ref):
    h = h_ref[:]
    H1 = jnp.dot(h, W1_ref[:], preferred_element_type=jnp.float32)
    H1 = H1 + b1_ref[:][None, :]
    m1 = jnp.mean(H1, axis=0, keepdims=True)
    v1 = jnp.mean((H1 - m1) ** 2, axis=0, keepdims=True)
    X1 = g1_ref[:][None, :] * (H1 - m1) * lax.rsqrt(v1 + 1e-5)
    X1 = jnp.maximum(X1 + be1_ref[:][None, :], 0.0)
    H2 = jnp.dot(X1, W2_ref[:], preferred_element_type=jnp.float32)
    H2 = H2 + b2_ref[:][None, :]
    m2 = jnp.mean(H2, axis=0, keepdims=True)
    v2 = jnp.mean((H2 - m2) ** 2, axis=0, keepdims=True)
    X2 = g2_ref[:][None, :] * (H2 - m2) * lax.rsqrt(v2 + 1e-5)
    X2 = jnp.maximum(X2 + be2_ref[:][None, :], 0.0)
    gmf = h[:, :_D] * h[:, _D:2 * _D]
    w = Wfc_ref[:]
    acc = jnp.dot(gmf, w[:_D, :], preferred_element_type=jnp.float32)
    acc = acc + jnp.dot(X2, w[_D:, :], preferred_element_type=jnp.float32)
    out_ref[:] = acc + bfc_ref[:][None, :]


def _tc_mlp(h2d, W1, b1, g1, be1, W2, b2, g2, be2, Wfc, bfc):
    B = h2d.shape[0]
    return pl.pallas_call(
        _mlp_body,
        out_shape=jax.ShapeDtypeStruct((B, 1), jnp.float32),
    )(h2d, W1, b1, g1, be1, W2, b2, g2, be2, Wfc, bfc)


def kernel(x, emb_table, W1, b1, g1, be1, W2, b2, g2, be2, Wfc, bfc):
    B = x.shape[0]
    offsets = jnp.array([0, emb_table.shape[0] // 2], dtype=x.dtype)
    idx = (x + offsets[None, :]).reshape(-1)
    idx3 = idx.reshape(_NW, (2 * B) // (_NW * _CHUNK), _CHUNK)
    rows = _sc_gather(emb_table, idx3)
    h2d = rows.reshape(B, 2 * _D)
    out = _tc_mlp(h2d, W1, b1, g1, be1, W2, b2, g2, be2, Wfc, bfc)
    return out.reshape(B)
